# baseline (speedup 1.0000x reference)
import functools

import jax
import jax.numpy as jnp
from jax import lax
from jax.experimental import pallas as pl
from jax.experimental.pallas import tpu as pltpu
from jax.experimental.pallas import tpu_sc as plsc

_NC = 2
_NS = 16
_NW = _NC * _NS
_LANES = 16
_SUBL = 8
_TC_TILE = 256
_TC_ROWS = 5632


def _tc_body(idx_ref, x_ref, o_ref):
    k = idx_ref.shape[-1]
    f = x_ref.shape[-1]
    iota = jax.lax.broadcasted_iota(jnp.int32, (f, k), 0)
    sel = (iota == idx_ref[...]).astype(jnp.bfloat16)
    o_ref[...] = jax.lax.dot_general(
        x_ref[...].astype(jnp.bfloat16), sel,
        (((1,), (0,)), ((), ())), preferred_element_type=jnp.float32,
    )


def _tc_gather(x2d, mask, rt, feat, k):
    return pl.pallas_call(
        _tc_body,
        grid=(rt // _TC_TILE,),
        in_specs=[
            pl.BlockSpec((1, k), lambda i: (0, 0)),
            pl.BlockSpec((_TC_TILE, feat), lambda i: (i, 0)),
        ],
        out_specs=pl.BlockSpec((_TC_TILE, k), lambda i: (i, 0)),
        out_shape=jax.ShapeDtypeStruct((rt, k), jnp.float32),
    )(mask.reshape(1, k), x2d)


def _sc_gather(x2d, mask, row_off, rs, feat, k):
    rpw = rs // _NW
    vregs = k // _LANES
    tpw = rpw // _SUBL
    mesh = plsc.VectorSubcoreMesh(
        core_axis_name="c", subcore_axis_name="s",
        num_cores=_NC, num_subcores=_NS,
    )

    @functools.partial(
        pl.kernel,
        out_type=jax.ShapeDtypeStruct((rs, k), jnp.float32),
        mesh=mesh,
        scratch_types=[
            pltpu.VMEM((k,), jnp.int32),
            pltpu.VMEM((_SUBL, feat), jnp.float32),
            pltpu.VMEM((_SUBL, feat), jnp.float32),
            pltpu.VMEM((rpw, k), jnp.float32),
            pltpu.SemaphoreType.DMA,
            pltpu.SemaphoreType.DMA,
        ],
        compiler_params=pltpu.CompilerParams(
            use_tc_tiling_on_sc=True, needs_layout_passes=False
        ),
    )
    def run(x_hbm, mask_hbm, out_hbm, mask_v, ch0, ch1, out_v, sem0, sem1):
        wid = lax.axis_index("s") * _NC + lax.axis_index("c")
        g_base = (row_off + wid * rpw) // _SUBL
        pltpu.sync_copy(mask_hbm, mask_v)
        cvec = [mask_v[pl.ds(j * _LANES, _LANES)] for j in range(vregs)]
        rvec = [
            jnp.full((_LANES,), rm, dtype=jnp.int32) for rm in range(_SUBL)
        ]

        def start(g, buf, sem):
            gi = lax.min(g, tpw - 1)
            pltpu.async_copy(
                x_hbm.at[pl.ds((g_base + gi) * _SUBL, _SUBL)], buf, sem
            )

        def wait(buf, sem):
            pltpu.make_async_copy(
                x_hbm.at[pl.ds(0, _SUBL)], buf, sem
            ).wait()

        def compact(g, buf):
            for rm in range(_SUBL):
                r = g * _SUBL + rm
                for j in range(vregs):
                    v = plsc.load_gather(buf, [rvec[rm], cvec[j]])
                    out_v[r, pl.ds(j * _LANES, _LANES)] = v

        start(0, ch0, sem0)

        def body(i, _):
            start(2 * i + 1, ch1, sem1)
            wait(ch0, sem0)
            compact(2 * i, ch0)
            start(2 * i + 2, ch0, sem0)
            wait(ch1, sem1)
            compact(2 * i + 1, ch1)
            return 0

        lax.fori_loop(0, tpw // 2, body, 0)
        wait(ch0, sem0)
        pltpu.sync_copy(out_v, out_hbm.at[pl.ds(wid * rpw, rpw)])

    return run(x2d, mask)


@jax.jit
def kernel(x, output_tensor_mask):
    b, s, f = x.shape
    k = output_tensor_mask.shape[0]
    rows = b * s
    x2d = x.reshape(rows, f)
    out_sc = _sc_gather(
        x2d, output_tensor_mask, _TC_ROWS, rows - _TC_ROWS, f, k
    )
    out_tc = _tc_gather(x2d, output_tensor_mask, _TC_ROWS, f, k)
    return jnp.concatenate([out_tc, out_sc], axis=0).reshape(b, s, k)


# speedup vs baseline: 1.0133x; 1.0133x over previous
import functools

import jax
import jax.numpy as jnp
from jax import lax
from jax.experimental import pallas as pl
from jax.experimental.pallas import tpu as pltpu
from jax.experimental.pallas import tpu_sc as plsc

_NC = 2
_NS = 16
_NW = _NC * _NS
_LANES = 16
_SUBL = 8
_TC_TILE = 256
_TC_ROWS = 4608


def _tc_body(idx_ref, x_ref, o_ref):
    k = idx_ref.shape[-1]
    f = x_ref.shape[-1]
    iota = jax.lax.broadcasted_iota(jnp.int32, (f, k), 0)
    sel = (iota == idx_ref[...]).astype(jnp.bfloat16)
    o_ref[...] = jax.lax.dot_general(
        x_ref[...].astype(jnp.bfloat16), sel,
        (((1,), (0,)), ((), ())), preferred_element_type=jnp.float32,
    )


def _tc_gather(x2d, mask, rt, feat, k):
    return pl.pallas_call(
        _tc_body,
        grid=(rt // _TC_TILE,),
        in_specs=[
            pl.BlockSpec((1, k), lambda i: (0, 0)),
            pl.BlockSpec((_TC_TILE, feat), lambda i: (i, 0)),
        ],
        out_specs=pl.BlockSpec((_TC_TILE, k), lambda i: (i, 0)),
        out_shape=jax.ShapeDtypeStruct((rt, k), jnp.float32),
    )(mask.reshape(1, k), x2d)


def _sc_gather(x2d, mask, row_off, rs, feat, k):
    rpw = rs // _NW
    vregs = k // _LANES
    tpw = rpw // _SUBL
    mesh = plsc.VectorSubcoreMesh(
        core_axis_name="c", subcore_axis_name="s",
        num_cores=_NC, num_subcores=_NS,
    )

    @functools.partial(
        pl.kernel,
        out_type=jax.ShapeDtypeStruct((rs, k), jnp.float32),
        mesh=mesh,
        scratch_types=[
            pltpu.VMEM((k,), jnp.int32),
            pltpu.VMEM((_SUBL, feat), jnp.float32),
            pltpu.VMEM((_SUBL, feat), jnp.float32),
            pltpu.VMEM((rpw, k), jnp.float32),
            pltpu.SemaphoreType.DMA,
            pltpu.SemaphoreType.DMA,
        ],
        compiler_params=pltpu.CompilerParams(
            use_tc_tiling_on_sc=True, needs_layout_passes=False
        ),
    )
    def run(x_hbm, mask_hbm, out_hbm, mask_v, ch0, ch1, out_v, sem0, sem1):
        wid = lax.axis_index("s") * _NC + lax.axis_index("c")
        g_base = (row_off + wid * rpw) // _SUBL
        pltpu.sync_copy(mask_hbm, mask_v)
        cvec = [mask_v[pl.ds(j * _LANES, _LANES)] for j in range(vregs)]
        rvec = [
            jnp.full((_LANES,), rm, dtype=jnp.int32) for rm in range(_SUBL)
        ]

        def start(g, buf, sem):
            gi = lax.min(g, tpw - 1)
            pltpu.async_copy(
                x_hbm.at[pl.ds((g_base + gi) * _SUBL, _SUBL)], buf, sem
            )

        def wait(buf, sem):
            pltpu.make_async_copy(
                x_hbm.at[pl.ds(0, _SUBL)], buf, sem
            ).wait()

        def compact(g, buf):
            for rm in range(_SUBL):
                r = g * _SUBL + rm
                for j in range(vregs):
                    v = plsc.load_gather(buf, [rvec[rm], cvec[j]])
                    out_v[r, pl.ds(j * _LANES, _LANES)] = v

        start(0, ch0, sem0)

        def body(i, _):
            start(2 * i + 1, ch1, sem1)
            wait(ch0, sem0)
            compact(2 * i, ch0)
            start(2 * i + 2, ch0, sem0)
            wait(ch1, sem1)
            compact(2 * i + 1, ch1)
            return 0

        lax.fori_loop(0, tpw // 2, body, 0)
        wait(ch0, sem0)
        pltpu.sync_copy(out_v, out_hbm.at[pl.ds(wid * rpw, rpw)])

    return run(x2d, mask)


@jax.jit
def kernel(x, output_tensor_mask):
    b, s, f = x.shape
    k = output_tensor_mask.shape[0]
    rows = b * s
    x2d = x.reshape(rows, f)
    out_sc = _sc_gather(
        x2d, output_tensor_mask, _TC_ROWS, rows - _TC_ROWS, f, k
    )
    out_tc = _tc_gather(x2d, output_tensor_mask, _TC_ROWS, f, k)
    return jnp.concatenate([out_tc, out_sc], axis=0).reshape(b, s, k)
